# baseline (speedup 1.0000x reference)
import jax
import jax.numpy as jnp
from jax import lax
from jax.experimental import pallas as pl
from jax.experimental.pallas import tpu as pltpu
from jax.experimental.pallas import tpu_sc as plsc

BATCH = 16384
EMBED_DIM = 64
H1 = 128
H2 = 64
BN_EPS = 1e-5

_INFO = plsc.get_sparse_core_info()
_NC = _INFO.num_cores
_NS = _INFO.num_subcores
_NW = _NC * _NS
_CHUNK = 128
_ROWS_PER_W = BATCH // _NW
_CHUNKS_PER_W = _ROWS_PER_W // _CHUNK


def _gather_body(idx_hbm, emb_hbm, x_hbm, idx_v, rows_v, sem):
    wid = lax.axis_index("s") * _NC + lax.axis_index("c")
    base_chunk = wid * _CHUNKS_PER_W
    pltpu.sync_copy(idx_hbm.at[pl.ds(base_chunk, _CHUNKS_PER_W)], idx_v)
    copies = [
        pltpu.async_copy(emb_hbm.at[idx_v.at[j]], rows_v.at[j], sem)
        for j in range(_CHUNKS_PER_W)
    ]
    for c in copies:
        c.wait()
    for j in range(_CHUNKS_PER_W):
        pltpu.sync_copy(
            rows_v.at[j], x_hbm.at[pl.ds((base_chunk + j) * _CHUNK, _CHUNK)]
        )


_gather = pl.kernel(
    _gather_body,
    out_type=jax.ShapeDtypeStruct((BATCH, EMBED_DIM), jnp.float32),
    mesh=plsc.VectorSubcoreMesh(core_axis_name="c", subcore_axis_name="s"),
    compiler_params=pltpu.CompilerParams(use_tc_tiling_on_sc=False),
    scratch_types=[
        pltpu.VMEM((_CHUNKS_PER_W, _CHUNK), jnp.int32),
        pltpu.VMEM((_CHUNKS_PER_W, _CHUNK, EMBED_DIM), jnp.float32),
        pltpu.SemaphoreType.DMA,
    ],
)


_BLK = 1024
_INV = 1.0 / (1.0 + BN_EPS) ** 0.5


def _mlp_body(x_ref, w1_ref, b1_ref, g1_ref, be1_ref, w2_ref, b2_ref, g2_ref,
              be2_ref, o_ref):
    x = x_ref[...]
    h = jnp.dot(x, w1_ref[...], preferred_element_type=jnp.float32) + b1_ref[...]
    h = jnp.maximum(h, 0.0)
    h = h * (_INV * g1_ref[...]) + be1_ref[...]
    o = jnp.dot(h, w2_ref[...], preferred_element_type=jnp.float32) + b2_ref[...]
    o = jnp.maximum(o, 0.0)
    o_ref[...] = o * (_INV * g2_ref[...]) + be2_ref[...]


def _full(shape):
    return pl.BlockSpec(shape, lambda i: (0,) * len(shape))


_mlp = pl.pallas_call(
    _mlp_body,
    grid=(BATCH // _BLK,),
    in_specs=[
        pl.BlockSpec((_BLK, EMBED_DIM), lambda i: (i, 0)),
        _full((EMBED_DIM, H1)),
        _full((1, H1)),
        _full((1, H1)),
        _full((1, H1)),
        _full((H1, H2)),
        _full((1, H2)),
        _full((1, H2)),
        _full((1, H2)),
    ],
    out_specs=pl.BlockSpec((_BLK, H2), lambda i: (i, 0)),
    out_shape=jax.ShapeDtypeStruct((BATCH, H2), jnp.float32),
)


@jax.jit
def kernel(user_ids, emb, W1, b1, g1, be1, W2, b2, g2, be2):
    idx = user_ids.astype(jnp.int32).reshape(_NW * _CHUNKS_PER_W, _CHUNK)
    x = _gather(idx, emb)
    return _mlp(
        x,
        W1,
        b1.reshape(1, H1),
        g1.reshape(1, H1),
        be1.reshape(1, H1),
        W2,
        b2.reshape(1, H2),
        g2.reshape(1, H2),
        be2.reshape(1, H2),
    )


# speedup vs baseline: 1.6895x; 1.6895x over previous
import jax
import jax.numpy as jnp
from jax import lax
from jax.experimental import pallas as pl
from jax.experimental.pallas import tpu as pltpu
from jax.experimental.pallas import tpu_sc as plsc

BATCH = 16384
EMBED_DIM = 64
H1 = 128
H2 = 64
BN_EPS = 1e-5

_INFO = plsc.get_sparse_core_info()
_NC = _INFO.num_cores
_NS = _INFO.num_subcores
_NW = _NC * _NS
_CHUNK = 128
_ROWS_PER_W = BATCH // _NW
_CHUNKS_PER_W = _ROWS_PER_W // _CHUNK


_K = 16


def _gather_body(idx_hbm, emb_hbm, x_hbm, idx_v, rows_v, sem):
    wid = lax.axis_index("s") * _NC + lax.axis_index("c")
    base = wid * _ROWS_PER_W
    pltpu.sync_copy(idx_hbm.at[pl.ds(base, _ROWS_PER_W)], idx_v)

    def fire(p, _):
        v = idx_v[pl.ds(p * _K, _K)]
        for k in range(_K):
            slot = p * _K + k
            pltpu.async_copy(
                emb_hbm.at[pl.ds(v[k], 1)], rows_v.at[pl.ds(slot, 1)], sem
            )
        return _

    lax.fori_loop(0, _ROWS_PER_W // _K, fire, 0)

    def drain(p, _):
        for k in range(_K):
            slot = p * _K + k
            pltpu.make_async_copy(
                emb_hbm.at[pl.ds(0, 1)], rows_v.at[pl.ds(slot, 1)], sem
            ).wait()
        return _

    lax.fori_loop(0, _ROWS_PER_W // _K, drain, 0)
    pltpu.sync_copy(rows_v, x_hbm.at[pl.ds(base, _ROWS_PER_W)])


_gather = pl.kernel(
    _gather_body,
    out_type=jax.ShapeDtypeStruct((BATCH, EMBED_DIM), jnp.float32),
    mesh=plsc.VectorSubcoreMesh(core_axis_name="c", subcore_axis_name="s"),
    scratch_types=[
        pltpu.VMEM((_ROWS_PER_W,), jnp.int32),
        pltpu.VMEM((_ROWS_PER_W, EMBED_DIM), jnp.float32),
        pltpu.SemaphoreType.DMA,
    ],
)


_BLK = 1024
_INV = 1.0 / (1.0 + BN_EPS) ** 0.5


def _mlp_body(x_ref, w1_ref, b1_ref, g1_ref, be1_ref, w2_ref, b2_ref, g2_ref,
              be2_ref, o_ref):
    x = x_ref[...]
    h = jnp.dot(x, w1_ref[...], preferred_element_type=jnp.float32) + b1_ref[...]
    h = jnp.maximum(h, 0.0)
    h = h * (_INV * g1_ref[...]) + be1_ref[...]
    o = jnp.dot(h, w2_ref[...], preferred_element_type=jnp.float32) + b2_ref[...]
    o = jnp.maximum(o, 0.0)
    o_ref[...] = o * (_INV * g2_ref[...]) + be2_ref[...]


def _full(shape):
    return pl.BlockSpec(shape, lambda i: (0,) * len(shape))


_mlp = pl.pallas_call(
    _mlp_body,
    grid=(BATCH // _BLK,),
    in_specs=[
        pl.BlockSpec((_BLK, EMBED_DIM), lambda i: (i, 0)),
        _full((EMBED_DIM, H1)),
        _full((1, H1)),
        _full((1, H1)),
        _full((1, H1)),
        _full((H1, H2)),
        _full((1, H2)),
        _full((1, H2)),
        _full((1, H2)),
    ],
    out_specs=pl.BlockSpec((_BLK, H2), lambda i: (i, 0)),
    out_shape=jax.ShapeDtypeStruct((BATCH, H2), jnp.float32),
)


@jax.jit
def kernel(user_ids, emb, W1, b1, g1, be1, W2, b2, g2, be2):
    idx = user_ids.astype(jnp.int32)
    x = _gather(idx, emb)
    return _mlp(
        x,
        W1,
        b1.reshape(1, H1),
        g1.reshape(1, H1),
        be1.reshape(1, H1),
        W2,
        b2.reshape(1, H2),
        g2.reshape(1, H2),
        be2.reshape(1, H2),
    )
